# baseline (speedup 1.0000x reference)
import functools

import jax
import jax.numpy as jnp
from jax import lax
from jax.experimental import pallas as pl
from jax.experimental.pallas import tpu as pltpu
from jax.experimental.pallas import tpu_sc as plsc

B = 4096
S = 200
E = 64
UNITS = 256

NC = 2
NS = 16
NW = NC * NS
BPW = B // NW
NBUF = 4
LANES = 16
EV = E // LANES

_sc_mesh = plsc.VectorSubcoreMesh(
    core_axis_name="c", subcore_axis_name="s", num_cores=NC, num_subcores=NS
)


@functools.partial(
    pl.kernel,
    out_type=jax.ShapeDtypeStruct((B, E), jnp.float32),
    mesh=_sc_mesh,
    scratch_types=[
        pltpu.VMEM((BPW, S), jnp.int32),
        pltpu.VMEM((NBUF, S, E), jnp.float32),
        pltpu.VMEM((BPW, E), jnp.float32),
        pltpu.SemaphoreType.DMA((NBUF,)),
    ],
    compiler_params=pltpu.CompilerParams(use_tc_tiling_on_sc=False),
)
def _pool_sc(x_hbm, table_hbm, out_hbm, idx_v, bufs, out_v, sems):
    wid = lax.axis_index("s") * NC + lax.axis_index("c")
    base = wid * BPW
    pltpu.sync_copy(x_hbm.at[pl.ds(base, BPW)], idx_v)

    inv_s = jnp.full((LANES,), 1.0 / S, dtype=jnp.float32)

    def issue(r, slot):
        pltpu.async_copy(table_hbm.at[idx_v.at[r]], bufs.at[slot], sems.at[slot])

    def wait(r, slot):
        pltpu.make_async_copy(
            table_hbm.at[idx_v.at[r]], bufs.at[slot], sems.at[slot]
        ).wait()

    for b in range(NBUF):
        issue(b, b)

    def row_body(r, _):
        slot = lax.rem(r, NBUF)
        nch = 4
        acc = [[jnp.zeros((LANES,), jnp.float32) for _ in range(nch)]
               for _ in range(EV)]
        wait(r, slot)
        for j in range(S):
            for k in range(EV):
                acc[k][j % nch] = (
                    acc[k][j % nch] + bufs[slot, j, pl.ds(k * LANES, LANES)]
                )
        issue(jnp.minimum(r + NBUF, BPW - 1), slot)
        for k in range(EV):
            tot = (acc[k][0] + acc[k][1]) + (acc[k][2] + acc[k][3])
            out_v[r, pl.ds(k * LANES, LANES)] = tot * inv_s
        return ()

    lax.fori_loop(0, BPW, row_body, ())
    for b in range(NBUF):
        wait(BPW - 1, b)
    pltpu.sync_copy(out_v, out_hbm.at[pl.ds(base, BPW)])


BT = 512


def _mlp_tc(pooled_ref, w1_ref, b1_ref, w2_ref, b2_ref, out_ref):
    h = jnp.maximum(
        jnp.dot(pooled_ref[:], w1_ref[:], preferred_element_type=jnp.float32)
        + b1_ref[:],
        0.0,
    )
    logit = jnp.sum(h * w2_ref[:], axis=1, keepdims=True) + b2_ref[:]
    out_ref[:] = jax.nn.sigmoid(logit)


def kernel(x, table, W1, b1, W2, b2):
    xi = x.astype(jnp.int32).reshape(B, S)
    pooled = _pool_sc(xi, table)

    grid = (B // BT,)
    out = pl.pallas_call(
        _mlp_tc,
        grid=grid,
        in_specs=[
            pl.BlockSpec((BT, E), lambda i: (i, 0)),
            pl.BlockSpec((E, UNITS), lambda i: (0, 0)),
            pl.BlockSpec((1, UNITS), lambda i: (0, 0)),
            pl.BlockSpec((1, UNITS), lambda i: (0, 0)),
            pl.BlockSpec((1, 1), lambda i: (0, 0)),
        ],
        out_specs=pl.BlockSpec((BT, 1), lambda i: (i, 0)),
        out_shape=jax.ShapeDtypeStruct((B, 1), jnp.float32),
    )(pooled, W1, b1.reshape(1, UNITS), W2.reshape(1, UNITS), b2.reshape(1, 1))
    return out


# speedup vs baseline: 2.4382x; 2.4382x over previous
import functools

import jax
import jax.numpy as jnp
from jax import lax
from jax.experimental import pallas as pl
from jax.experimental.pallas import tpu as pltpu
from jax.experimental.pallas import tpu_sc as plsc

B = 4096
S = 200
E = 64
UNITS = 256

NC = 2
NS = 16
NW = NC * NS
BPW = B // NW
NBUF = 4
LANES = 16
EV = E // LANES

_sc_mesh = plsc.VectorSubcoreMesh(
    core_axis_name="c", subcore_axis_name="s", num_cores=NC, num_subcores=NS
)


@functools.partial(
    pl.kernel,
    out_type=jax.ShapeDtypeStruct((B, E), jnp.float32),
    mesh=_sc_mesh,
    scratch_types=[
        pltpu.VMEM((BPW, S), jnp.int32),
        pltpu.VMEM((NBUF, S, E), jnp.float32),
        pltpu.VMEM((BPW, E), jnp.float32),
        pltpu.SemaphoreType.DMA((NBUF,)),
    ],
    compiler_params=pltpu.CompilerParams(use_tc_tiling_on_sc=False),
)
def _pool_sc(x_hbm, table_hbm, out_hbm, idx_v, bufs, out_v, sems):
    wid = lax.axis_index("s") * NC + lax.axis_index("c")
    base = wid * BPW
    pltpu.sync_copy(x_hbm.at[pl.ds(base, BPW)], idx_v)

    inv_s = jnp.full((LANES,), 1.0 / S, dtype=jnp.float32)

    def issue(r, slot):
        pltpu.async_copy(table_hbm.at[idx_v.at[r]], bufs.at[slot], sems.at[slot])

    def wait(r, slot):
        pltpu.make_async_copy(
            table_hbm.at[idx_v.at[r]], bufs.at[slot], sems.at[slot]
        ).wait()

    for b in range(NBUF):
        issue(b, b)

    zero16 = jnp.zeros((LANES,), jnp.float32)

    def row_body(r, _):
        slot = lax.rem(r, NBUF)
        wait(r, slot)
        init = (tuple(zero16 for _ in range(EV)),
                tuple(zero16 for _ in range(EV)))

        @plsc.parallel_loop(0, S, step=2, unroll=8, carry=init)
        def jloop(j, carry):
            acc_a, acc_b = carry
            new_a = tuple(
                acc_a[k] + bufs[slot, j, pl.ds(k * LANES, LANES)]
                for k in range(EV)
            )
            new_b = tuple(
                acc_b[k] + bufs[slot, j + 1, pl.ds(k * LANES, LANES)]
                for k in range(EV)
            )
            return (new_a, new_b)

        acc_a, acc_b = jloop
        issue(jnp.minimum(r + NBUF, BPW - 1), slot)
        for k in range(EV):
            out_v[r, pl.ds(k * LANES, LANES)] = (acc_a[k] + acc_b[k]) * inv_s
        return ()

    lax.fori_loop(0, BPW, row_body, ())
    for b in range(NBUF):
        wait(BPW - 1, b)
    pltpu.sync_copy(out_v, out_hbm.at[pl.ds(base, BPW)])


BT = 512


def _mlp_tc(pooled_ref, w1_ref, b1_ref, w2_ref, b2_ref, out_ref):
    h = jnp.maximum(
        jnp.dot(pooled_ref[:], w1_ref[:], preferred_element_type=jnp.float32)
        + b1_ref[:],
        0.0,
    )
    logit = jnp.sum(h * w2_ref[:], axis=1, keepdims=True) + b2_ref[:]
    out_ref[:] = jax.nn.sigmoid(logit)


def kernel(x, table, W1, b1, W2, b2):
    xi = x.astype(jnp.int32).reshape(B, S)
    pooled = _pool_sc(xi, table)

    grid = (B // BT,)
    out = pl.pallas_call(
        _mlp_tc,
        grid=grid,
        in_specs=[
            pl.BlockSpec((BT, E), lambda i: (i, 0)),
            pl.BlockSpec((E, UNITS), lambda i: (0, 0)),
            pl.BlockSpec((1, UNITS), lambda i: (0, 0)),
            pl.BlockSpec((1, UNITS), lambda i: (0, 0)),
            pl.BlockSpec((1, 1), lambda i: (0, 0)),
        ],
        out_specs=pl.BlockSpec((BT, 1), lambda i: (i, 0)),
        out_shape=jax.ShapeDtypeStruct((B, 1), jnp.float32),
    )(pooled, W1, b1.reshape(1, UNITS), W2.reshape(1, UNITS), b2.reshape(1, 1))
    return out


# speedup vs baseline: 2.4401x; 1.0008x over previous
import functools

import jax
import jax.numpy as jnp
from jax import lax
from jax.experimental import pallas as pl
from jax.experimental.pallas import tpu as pltpu
from jax.experimental.pallas import tpu_sc as plsc

B = 4096
S = 200
E = 64
UNITS = 256

NC = 2
NS = 16
NW = NC * NS
BPW = B // NW
NBUF = 4
LANES = 16
EV = E // LANES

_sc_mesh = plsc.VectorSubcoreMesh(
    core_axis_name="c", subcore_axis_name="s", num_cores=NC, num_subcores=NS
)


@functools.partial(
    pl.kernel,
    out_type=jax.ShapeDtypeStruct((B * E,), jnp.float32),
    mesh=_sc_mesh,
    scratch_types=[
        pltpu.VMEM((BPW * S,), jnp.int32),
        pltpu.VMEM((NBUF, S, E), jnp.float32),
        pltpu.VMEM((BPW * E,), jnp.float32),
        pltpu.SemaphoreType.DMA((NBUF,)),
    ],
    compiler_params=pltpu.CompilerParams(use_tc_tiling_on_sc=False),
)
def _pool_sc(x_hbm, table_hbm, out_hbm, idx_v, bufs, out_v, sems):
    wid = lax.axis_index("s") * NC + lax.axis_index("c")
    base = wid * BPW
    pltpu.sync_copy(x_hbm.at[pl.ds(base * S, BPW * S)], idx_v)

    inv_s = jnp.full((LANES,), 1.0 / S, dtype=jnp.float32)

    def issue(r, slot):
        pltpu.async_copy(
            table_hbm.at[idx_v.at[pl.ds(r * S, S)]], bufs.at[slot], sems.at[slot]
        )

    def wait(r, slot):
        pltpu.make_async_copy(
            table_hbm.at[idx_v.at[pl.ds(r * S, S)]], bufs.at[slot], sems.at[slot]
        ).wait()

    for b in range(NBUF):
        issue(b, b)

    zero16 = jnp.zeros((LANES,), jnp.float32)

    def row_body(r, _):
        slot = lax.rem(r, NBUF)
        wait(r, slot)
        init = (tuple(zero16 for _ in range(EV)),
                tuple(zero16 for _ in range(EV)))

        @plsc.parallel_loop(0, S, step=2, unroll=8, carry=init)
        def jloop(j, carry):
            acc_a, acc_b = carry
            new_a = tuple(
                acc_a[k] + bufs[slot, j, pl.ds(k * LANES, LANES)]
                for k in range(EV)
            )
            new_b = tuple(
                acc_b[k] + bufs[slot, j + 1, pl.ds(k * LANES, LANES)]
                for k in range(EV)
            )
            return (new_a, new_b)

        acc_a, acc_b = jloop
        issue(jnp.minimum(r + NBUF, BPW - 1), slot)
        for k in range(EV):
            out_v[pl.ds(r * E + k * LANES, LANES)] = (acc_a[k] + acc_b[k]) * inv_s
        return ()

    lax.fori_loop(0, BPW, row_body, ())
    for b in range(NBUF):
        wait(BPW - 1, b)
    pltpu.sync_copy(out_v, out_hbm.at[pl.ds(base * E, BPW * E)])


BT = 512


def _mlp_tc(pooled_ref, w1_ref, b1_ref, w2_ref, b2_ref, out_ref):
    h = jnp.maximum(
        jnp.dot(pooled_ref[:], w1_ref[:], preferred_element_type=jnp.float32)
        + b1_ref[:],
        0.0,
    )
    logit = jnp.sum(h * w2_ref[:], axis=1, keepdims=True) + b2_ref[:]
    out_ref[:] = jax.nn.sigmoid(logit)


def kernel(x, table, W1, b1, W2, b2):
    xi = x.astype(jnp.int32).reshape(B * S)
    pooled = _pool_sc(xi, table).reshape(B, E)

    grid = (B // BT,)
    out = pl.pallas_call(
        _mlp_tc,
        grid=grid,
        in_specs=[
            pl.BlockSpec((BT, E), lambda i: (i, 0)),
            pl.BlockSpec((E, UNITS), lambda i: (0, 0)),
            pl.BlockSpec((1, UNITS), lambda i: (0, 0)),
            pl.BlockSpec((1, UNITS), lambda i: (0, 0)),
            pl.BlockSpec((1, 1), lambda i: (0, 0)),
        ],
        out_specs=pl.BlockSpec((BT, 1), lambda i: (i, 0)),
        out_shape=jax.ShapeDtypeStruct((B, 1), jnp.float32),
    )(pooled, W1, b1.reshape(1, UNITS), W2.reshape(1, UNITS), b2.reshape(1, 1))
    return out
